# baseline (speedup 1.0000x reference)
import jax
import jax.numpy as jnp
from jax.experimental import pallas as pl
from jax.experimental.pallas import tpu as pltpu


def _vq_kernel(z_ref, w_ref, out_ref, zs_ref, wn_ref, w2_ref):
    @pl.when(pl.program_id(0) == 0)
    def _init():
        wn = w_ref[...] * -2.0
        wn_ref[...] = wn
        w2_ref[:, 0:1] = jnp.sum(wn * wn, axis=1, keepdims=True) * 0.25

    zs_ref[...] = z_ref[0].reshape(64, 4096)
    s = jax.lax.dot_general(
        wn_ref[...], zs_ref[...], (((1,), (0,)), ((), ())),
        preferred_element_type=jnp.float32)
    d2 = s + w2_ref[:, 0:1]
    out_ref[0, 0, :] = jnp.argmin(d2, axis=0).astype(jnp.int32)


def kernel(z, W):
    t, a, b, c = z.shape
    k = W.shape[0]
    return pl.pallas_call(
        _vq_kernel,
        grid=(t,),
        in_specs=[
            pl.BlockSpec((1, a, b, c), lambda i: (i, 0, 0, 0)),
            pl.BlockSpec((k, a), lambda i: (0, 0)),
        ],
        out_specs=pl.BlockSpec((1, 1, b * c), lambda i: (i, 0, 0)),
        out_shape=jax.ShapeDtypeStruct((t, 1, b * c), jnp.int32),
        scratch_shapes=[
            pltpu.VMEM((a, b * c), jnp.float32),
            pltpu.VMEM((k, a), jnp.float32),
            pltpu.VMEM((k, 128), jnp.float32),
        ],
        compiler_params=pltpu.CompilerParams(
            dimension_semantics=("arbitrary",)),
    )(z, W).reshape(t, b, c)
